# Initial kernel scaffold; baseline (speedup 1.0000x reference)
import jax
import jax.numpy as jnp
from jax.experimental import pallas as pl


def kernel(x, w1, b1, w2, b2):
    raise NotImplementedError("write your pallas kernel here")


# speedup vs baseline: 1.1583x; 1.1583x over previous
import jax
import jax.numpy as jnp
from jax.experimental import pallas as pl
from jax.experimental.pallas import tpu as pltpu


def _gate_kernel(x_ref, w1t_ref, b1_ref, w2t_ref, b2_ref, o_ref):
    x = x_ref[...]

    p_avg = jnp.mean(x, axis=-1, dtype=jnp.float32)
    p_max = jnp.max(x, axis=-1).astype(jnp.float32)

    p = jnp.concatenate([p_avg, p_max], axis=0)
    h = jnp.dot(p, w1t_ref[...], preferred_element_type=jnp.float32) + b1_ref[...]
    h = jnp.maximum(h, 0.0)
    a = jnp.dot(h, w2t_ref[...], preferred_element_type=jnp.float32) + b2_ref[...]
    att = a[:1] + a[1:]

    scale = jax.nn.sigmoid(att)
    o_ref[...] = (x * scale[:, :, None]).astype(o_ref.dtype)


def kernel(x, w1, b1, w2, b2):
    N, C, D, H, W = x.shape
    S = D * H * W
    Ch = w1.shape[0]

    w1t = jnp.asarray(w1, jnp.float32).T
    w2t = jnp.asarray(w2, jnp.float32).T
    b1r = jnp.asarray(b1, jnp.float32).reshape(1, Ch)
    b2r = jnp.asarray(b2, jnp.float32).reshape(1, C)

    x_flat = x.reshape(N, C, S)

    item = jnp.dtype(x.dtype).itemsize
    block_bytes = C * S * item
    limit = min(4 * block_bytes + (2 << 20), 60 * 1024 * 1024)

    out_flat = pl.pallas_call(
        _gate_kernel,
        out_shape=jax.ShapeDtypeStruct((N, C, S), x.dtype),
        grid=(N,),
        in_specs=[
            pl.BlockSpec((1, C, S), lambda n: (n, 0, 0)),
            pl.BlockSpec((C, Ch), lambda n: (0, 0)),
            pl.BlockSpec((1, Ch), lambda n: (0, 0)),
            pl.BlockSpec((Ch, C), lambda n: (0, 0)),
            pl.BlockSpec((1, C), lambda n: (0, 0)),
        ],
        out_specs=pl.BlockSpec((1, C, S), lambda n: (n, 0, 0)),
        compiler_params=pltpu.CompilerParams(
            dimension_semantics=("parallel",),
            vmem_limit_bytes=int(limit),
        ),
    )(x_flat, w1t, b1r, w2t, b2r)
    return out_flat.reshape(N, C, D, H, W)
